# baseline (speedup 1.0000x reference)
import functools

import jax
import jax.numpy as jnp
from jax import lax
from jax.experimental import pallas as pl
from jax.experimental.pallas import tpu as pltpu
from jax.experimental.pallas import tpu_sc as plsc

EMBEDDING_DIM = 128
NUM_POS = 8192
PAD_IDX = 1
B, S = 4, 8192
N_TOK = B * S

NC, NS, L = 2, 16, 16
NW = NC * NS
TOK_PER_W = N_TOK // NW
W_PER_ROW = S // TOK_PER_W
VECS = TOK_PER_W // L
CHUNK = 128
CHUNKS = TOK_PER_W // CHUNK
NBUF = 7

_sc_mesh = plsc.VectorSubcoreMesh(core_axis_name="c", subcore_axis_name="s")


@functools.partial(
    pl.kernel,
    mesh=_sc_mesh,
    out_type=jax.ShapeDtypeStruct((N_TOK, EMBEDDING_DIM), jnp.float32),
    scratch_types=[
        pltpu.VMEM((TOK_PER_W,), jnp.int32),
        pltpu.VMEM((TOK_PER_W,), jnp.int32),
        pltpu.VMEM((TOK_PER_W,), jnp.int32),
        pltpu.VMEM((L,), jnp.int32),
        pltpu.VMEM((NS, L), jnp.int32),
        pltpu.VMEM_SHARED((NS, L), jnp.int32),
        *[pltpu.VMEM((CHUNK, EMBEDDING_DIM), jnp.float32) for _ in range(NBUF)],
        *[pltpu.SemaphoreType.DMA for _ in range(2 * NBUF)],
    ],
)
def _pse_kernel(batch_hbm, seg_hbm, weight_hbm, out_hbm, bat_v, seg_v, idx_v,
                my_tot, tot_v, tot_sh, *bufs_and_sems):
    rows = bufs_and_sems[:NBUF]
    gsem = bufs_and_sems[NBUF : 2 * NBUF]
    wsem = bufs_and_sems[2 * NBUF :]
    c = lax.axis_index("c")
    s = lax.axis_index("s")
    row_b = 2 * c + s // W_PER_ROW
    p = s % W_PER_ROW
    col0 = p * TOK_PER_W
    base = row_b * S + col0

    pltpu.sync_copy(batch_hbm.at[row_b, pl.ds(col0, TOK_PER_W)], bat_v)
    pltpu.sync_copy(seg_hbm.at[row_b, pl.ds(col0, TOK_PER_W)], seg_v)

    lane = lax.iota(jnp.int32, L)

    gd = lax.GatherDimensionNumbers(
        offset_dims=(), collapsed_slice_dims=(0,), start_index_map=(0,)
    )

    def lane_gather(y, src):
        return lax.gather(
            y, src[:, None], gd, slice_sizes=(1,),
            mode=lax.GatherScatterMode.PROMISE_IN_BOUNDS,
        )

    def vec_cumsum(x):
        y = x
        for k in (1, 2, 4, 8):
            src = jnp.maximum(lane - k, 0)
            sh = lane_gather(y, src)
            y = y + jnp.where(lane >= k, sh, jnp.zeros((L,), jnp.int32))
        return y

    lastv = jnp.full((L,), L - 1, jnp.int32)
    carry = jnp.zeros((L,), jnp.int32)
    for i in range(VECS):
        bb = bat_v[pl.ds(i * L, L)]
        m = jnp.minimum(jnp.abs(bb - PAD_IDX), jnp.int32(1))
        cs = vec_cumsum(m)
        idx_v[pl.ds(i * L, L)] = cs + carry
        carry = carry + lane_gather(cs, lastv)
    total = carry

    my_tot[...] = total
    pltpu.sync_copy(my_tot, tot_sh.at[s])
    plsc.subcore_barrier()
    pltpu.sync_copy(tot_sh, tot_v)

    g0 = (s // W_PER_ROW) * W_PER_ROW
    offv = jnp.zeros((L,), jnp.int32)
    for q in range(W_PER_ROW - 1):
        tv = tot_v[g0 + q]
        offv = offv + jnp.where(q < p, tv, jnp.zeros((L,), jnp.int32))

    for i in range(VECS):
        bb = bat_v[pl.ds(i * L, L)]
        sg = seg_v[pl.ds(i * L, L)]
        lc = idx_v[pl.ds(i * L, L)]
        mask = bb != PAD_IDX
        idx_v[pl.ds(i * L, L)] = jnp.where(
            mask, lc + offv + PAD_IDX + NUM_POS * sg, PAD_IDX
        )

    def g_start(j):
        pltpu.async_copy(
            weight_hbm.at[idx_v.at[pl.ds(j * CHUNK, CHUNK)]],
            rows[j % NBUF],
            gsem[j % NBUF],
        )

    def g_wait(j):
        pltpu.make_async_copy(
            weight_hbm.at[idx_v.at[pl.ds(j * CHUNK, CHUNK)]],
            rows[j % NBUF],
            gsem[j % NBUF],
        ).wait()

    def w_start(j):
        pltpu.async_copy(
            rows[j % NBUF],
            out_hbm.at[pl.ds(base + j * CHUNK, CHUNK)],
            wsem[j % NBUF],
        )

    def w_wait(j):
        pltpu.make_async_copy(
            rows[j % NBUF],
            out_hbm.at[pl.ds(base + j * CHUNK, CHUNK)],
            wsem[j % NBUF],
        ).wait()

    for j in range(NBUF - 1):
        g_start(j)
    for j in range(CHUNKS):
        g_wait(j)
        w_start(j)
        if j + NBUF - 1 < CHUNKS:
            if j >= 1:
                w_wait(j - 1)
            g_start(j + NBUF - 1)
    for j in range(CHUNKS - NBUF, CHUNKS):
        w_wait(j)


def kernel(batch, seg_labels, weight):
    batch = batch.astype(jnp.int32)
    seg_labels = seg_labels.astype(jnp.int32)
    out = _pse_kernel(batch, seg_labels, weight)
    return out.reshape(B, S, EMBEDDING_DIM)


# speedup vs baseline: 1.0386x; 1.0386x over previous
import functools

import jax
import jax.numpy as jnp
from jax import lax
from jax.experimental import pallas as pl
from jax.experimental.pallas import tpu as pltpu
from jax.experimental.pallas import tpu_sc as plsc

EMBEDDING_DIM = 128
NUM_POS = 8192
PAD_IDX = 1
B, S = 4, 8192
N_TOK = B * S

NC, NS = 2, 16
NW = NC * NS
TOK_PER_W = N_TOK // NW
W_PER_ROW = S // TOK_PER_W
CHUNK = 128
CHUNKS = TOK_PER_W // CHUNK
GPB = 2
NBUF = 3
WCHUNKS = CHUNKS // GPB


def _idx_body(batch_ref, seg_ref, idx_ref):
    b = batch_ref[...]
    seg = seg_ref[...]
    mask = b != PAD_IDX
    m = mask.astype(jnp.int32)
    c = m
    shift = 1
    while shift < S:
        shifted = jnp.concatenate(
            [jnp.zeros((B, shift), jnp.int32), c[:, : S - shift]], axis=1
        )
        c = c + shifted
        shift *= 2
    positions = c * m + PAD_IDX
    idx_ref[...] = jnp.where(mask, positions + NUM_POS * seg, PAD_IDX)


_idx_call = pl.pallas_call(
    _idx_body,
    out_shape=jax.ShapeDtypeStruct((B, S), jnp.int32),
)


_sc_mesh = plsc.VectorSubcoreMesh(core_axis_name="c", subcore_axis_name="s")


@functools.partial(
    pl.kernel,
    mesh=_sc_mesh,
    out_type=jax.ShapeDtypeStruct((N_TOK, EMBEDDING_DIM), jnp.float32),
    scratch_types=[
        pltpu.VMEM((TOK_PER_W,), jnp.int32),
        *[pltpu.VMEM((GPB * CHUNK, EMBEDDING_DIM), jnp.float32) for _ in range(NBUF)],
        *[pltpu.SemaphoreType.DMA for _ in range(2 * NBUF)],
    ],
)
def _gather_kernel(weight_hbm, idx_hbm, out_hbm, idx_v, *bufs_and_sems):
    rows = bufs_and_sems[:NBUF]
    gsem = bufs_and_sems[NBUF : 2 * NBUF]
    wsem = bufs_and_sems[2 * NBUF :]
    wid = lax.axis_index("s") * NC + lax.axis_index("c")
    row_b = wid // W_PER_ROW
    col0 = (wid % W_PER_ROW) * TOK_PER_W
    base = wid * TOK_PER_W
    pltpu.sync_copy(idx_hbm.at[row_b, pl.ds(col0, TOK_PER_W)], idx_v)

    def g_start(w, h):
        pltpu.async_copy(
            weight_hbm.at[idx_v.at[pl.ds((w * GPB + h) * CHUNK, CHUNK)]],
            rows[w % NBUF].at[pl.ds(h * CHUNK, CHUNK)],
            gsem[w % NBUF],
        )

    def g_wait_all(w):
        for h in range(GPB):
            pltpu.make_async_copy(
                weight_hbm.at[idx_v.at[pl.ds((w * GPB + h) * CHUNK, CHUNK)]],
                rows[w % NBUF].at[pl.ds(h * CHUNK, CHUNK)],
                gsem[w % NBUF],
            ).wait()

    def w_start(w):
        pltpu.async_copy(
            rows[w % NBUF],
            out_hbm.at[pl.ds(base + w * GPB * CHUNK, GPB * CHUNK)],
            wsem[w % NBUF],
        )

    def w_wait(w):
        pltpu.make_async_copy(
            rows[w % NBUF],
            out_hbm.at[pl.ds(base + w * GPB * CHUNK, GPB * CHUNK)],
            wsem[w % NBUF],
        ).wait()

    for w in range(NBUF - 1):
        for h in range(GPB):
            g_start(w, h)
    for w in range(WCHUNKS):
        g_wait_all(w)
        w_start(w)
        if w + NBUF - 1 < WCHUNKS:
            if w >= 1:
                w_wait(w - 1)
            for h in range(GPB):
                g_start(w + NBUF - 1, h)
    for w in range(max(WCHUNKS - NBUF, 0), WCHUNKS):
        w_wait(w)


def kernel(batch, seg_labels, weight):
    batch = batch.astype(jnp.int32)
    seg_labels = seg_labels.astype(jnp.int32)
    idx = _idx_call(batch, seg_labels)
    out = _gather_kernel(weight, idx)
    return out.reshape(B, S, EMBEDDING_DIM)


# speedup vs baseline: 1.0832x; 1.0429x over previous
import functools

import jax
import jax.numpy as jnp
from jax import lax
from jax.experimental import pallas as pl
from jax.experimental.pallas import tpu as pltpu
from jax.experimental.pallas import tpu_sc as plsc

EMBEDDING_DIM = 128
NUM_POS = 8192
PAD_IDX = 1
B, S = 4, 8192
N_TOK = B * S

NC, NS = 2, 16
NW = NC * NS
TOK_PER_W = N_TOK // NW
W_PER_ROW = S // TOK_PER_W
CHUNK = 128
CHUNKS = TOK_PER_W // CHUNK
NBUF = 7


def _idx_body(batch_ref, seg_ref, idx_ref):
    b = batch_ref[...]
    seg = seg_ref[...]
    mask = b != PAD_IDX
    m = mask.astype(jnp.int32)
    c = m
    shift = 1
    while shift < S:
        shifted = jnp.concatenate(
            [jnp.zeros((B, shift), jnp.int32), c[:, : S - shift]], axis=1
        )
        c = c + shifted
        shift *= 2
    positions = c * m + PAD_IDX
    idx_ref[...] = jnp.where(mask, positions + NUM_POS * seg, PAD_IDX)


_idx_call = pl.pallas_call(
    _idx_body,
    out_shape=jax.ShapeDtypeStruct((B, S), jnp.int32),
)


_sc_mesh = plsc.VectorSubcoreMesh(core_axis_name="c", subcore_axis_name="s")


@functools.partial(
    pl.kernel,
    mesh=_sc_mesh,
    out_type=jax.ShapeDtypeStruct((N_TOK, EMBEDDING_DIM), jnp.float32),
    scratch_types=[
        pltpu.VMEM((TOK_PER_W,), jnp.int32),
        *[pltpu.VMEM((CHUNK, EMBEDDING_DIM), jnp.float32) for _ in range(NBUF)],
        *[pltpu.SemaphoreType.DMA for _ in range(2 * NBUF + 1)],
    ],
)
def _gather_kernel(weight_hbm, idx_hbm, out_hbm, idx_v, *bufs_and_sems):
    rows = bufs_and_sems[:NBUF]
    gsem = bufs_and_sems[NBUF : 2 * NBUF]
    wsem = bufs_and_sems[2 * NBUF : 3 * NBUF]
    isem = bufs_and_sems[3 * NBUF]
    wid = lax.axis_index("s") * NC + lax.axis_index("c")
    row_b = wid // W_PER_ROW
    col0 = (wid % W_PER_ROW) * TOK_PER_W
    base = wid * TOK_PER_W
    HALF = TOK_PER_W // 2
    ih = [
        pltpu.make_async_copy(
            idx_hbm.at[row_b, pl.ds(col0 + h * HALF, HALF)],
            idx_v.at[pl.ds(h * HALF, HALF)],
            isem,
        )
        for h in range(2)
    ]
    ih[0].start()
    ih[1].start()
    ih[0].wait()

    def g_start(j):
        pltpu.async_copy(
            weight_hbm.at[idx_v.at[pl.ds(j * CHUNK, CHUNK)]],
            rows[j % NBUF],
            gsem[j % NBUF],
        )

    def g_wait(j):
        pltpu.make_async_copy(
            weight_hbm.at[idx_v.at[pl.ds(j * CHUNK, CHUNK)]],
            rows[j % NBUF],
            gsem[j % NBUF],
        ).wait()

    def w_start(j):
        pltpu.async_copy(
            rows[j % NBUF],
            out_hbm.at[pl.ds(base + j * CHUNK, CHUNK)],
            wsem[j % NBUF],
        )

    def w_wait(j):
        pltpu.make_async_copy(
            rows[j % NBUF],
            out_hbm.at[pl.ds(base + j * CHUNK, CHUNK)],
            wsem[j % NBUF],
        ).wait()

    for j in range(CHUNKS // 2):
        g_start(j)
    ih[1].wait()
    for j in range(CHUNKS // 2, NBUF - 1):
        g_start(j)
    for j in range(CHUNKS):
        g_wait(j)
        w_start(j)
        if j + NBUF - 1 < CHUNKS:
            if j >= 1:
                w_wait(j - 1)
            g_start(j + NBUF - 1)
    for j in range(CHUNKS - NBUF, CHUNKS):
        w_wait(j)


def kernel(batch, seg_labels, weight):
    batch = batch.astype(jnp.int32)
    seg_labels = seg_labels.astype(jnp.int32)
    idx = _idx_call(batch, seg_labels)
    out = _gather_kernel(weight, idx)
    return out.reshape(B, S, EMBEDDING_DIM)
